# baseline (speedup 1.0000x reference)
import functools

import jax
import jax.numpy as jnp
from jax import lax
from jax.experimental import pallas as pl
from jax.experimental.pallas import tpu as pltpu
from jax.experimental.pallas import tpu_sc as plsc

EMBED = 64
NUM_CORES = 2
NUM_SUBCORES = 16
NW = NUM_CORES * NUM_SUBCORES
CHUNK = 128
NBUF = 5


@functools.cache
def _make_gather2(nchunk):
    assert nchunk % NBUF == 0 and nchunk > NBUF
    mesh = plsc.VectorSubcoreMesh(core_axis_name="c", subcore_axis_name="s")
    out_t = jax.ShapeDtypeStruct((nchunk, EMBED, NW * CHUNK), jnp.float32)

    @functools.partial(
        pl.kernel,
        mesh=mesh,
        out_type=(out_t, out_t),
        compiler_params=pltpu.CompilerParams(
            use_tc_tiling_on_sc=False, needs_layout_passes=False),
        scratch_types=[
            pltpu.VMEM((nchunk, CHUNK), jnp.int32),
            pltpu.VMEM((NBUF, CHUNK, EMBED), jnp.float32),
            pltpu.VMEM((NBUF, EMBED, CHUNK), jnp.float32),
        ]
        + [pltpu.SemaphoreType.DMA] * (2 * NBUF),
    )
    def gather2(num_idx, unit_idx, num_tab, unit_tab, out_num, out_unit,
                idx_v, rows_v, st_v, *sems):
        wid = lax.axis_index("s") * NUM_CORES + lax.axis_index("c")
        sem_g = sems[:NBUF]
        sem_w = sems[NBUF:]

        lane = lax.broadcasted_iota(jnp.int32, (16,), 0)
        row_idx = [lane + 16 * g for g in range(CHUNK // 16)]

        def fire(tab, b, c):
            pltpu.async_copy(tab.at[idx_v.at[c]], rows_v.at[b], sem_g[b])

        def drain(tab, b, c):
            pltpu.make_async_copy(
                tab.at[idx_v.at[c]], rows_v.at[b], sem_g[b]).wait()

        def transpose(b):
            bv = lane * 0 + b

            def tr_step(i, colv):
                for k in range(4):
                    f = i * 4 + k
                    for g in range(CHUNK // 16):
                        vals = plsc.load_gather(
                            rows_v, [bv, row_idx[g], colv])
                        st_v[b, f, pl.ds(16 * g, 16)] = vals
                    colv = colv + 1
                return colv

            lax.fori_loop(0, EMBED // 4, tr_step, lane * 0)

        def put(out, b, c):
            pltpu.async_copy(
                st_v.at[b], out.at[c, :, pl.ds(wid * CHUNK, CHUNK)],
                sem_w[b])

        def put_wait(out, b, c):
            pltpu.make_async_copy(
                st_v.at[b], out.at[c, :, pl.ds(wid * CHUNK, CHUNK)],
                sem_w[b]).wait()

        def run_table(idx_hbm, tab, out):
            pltpu.sync_copy(idx_hbm.at[pl.ds(wid * nchunk, nchunk)], idx_v)
            for c in range(NBUF - 1):
                fire(tab, c, c)

            def step(i, carry):
                for b in range(NBUF):
                    c = i * NBUF + b
                    drain(tab, b, c)

                    @pl.when(c + NBUF - 1 < nchunk)
                    def _():
                        fire(tab, (b + NBUF - 1) % NBUF, c + NBUF - 1)

                    @pl.when(c >= NBUF)
                    def _():
                        put_wait(out, b, c - NBUF)

                    transpose(b)
                    put(out, b, c)
                return carry

            lax.fori_loop(0, nchunk // NBUF, step, 0)
            for c in range(nchunk - NBUF, nchunk):
                put_wait(out, c % NBUF, c)

        run_table(num_idx, num_tab, out_num)
        run_table(unit_idx, unit_tab, out_unit)

    return gather2


def kernel(num_tokens, unit_tokens, num_table, unit_table):
    B, S = num_tokens.shape
    assert B == NW * CHUNK and S % NBUF == 0

    def _arrange(tok):
        t = jnp.transpose(tok, (1, 0)).reshape(S, NW, CHUNK)
        return jnp.transpose(t, (1, 0, 2)).reshape(NW * S, CHUNK)

    ni = _arrange(num_tokens).astype(jnp.int32)
    ui = _arrange(unit_tokens).astype(jnp.int32)
    out_num, out_unit = _make_gather2(S)(ni, ui, num_table, unit_table)
    return (jnp.transpose(out_num, (2, 0, 1)),
            jnp.transpose(out_unit, (2, 0, 1)))


# speedup vs baseline: 1.9482x; 1.9482x over previous
import functools

import jax
import jax.numpy as jnp
from jax import lax
from jax.experimental import pallas as pl
from jax.experimental.pallas import tpu as pltpu
from jax.experimental.pallas import tpu_sc as plsc

EMBED = 64
NUM_CORES = 2
NUM_SUBCORES = 16
NW = NUM_CORES * NUM_SUBCORES
CHUNK = 128
NBUF = 5


@functools.cache
def _make_gather2(nchunk):
    assert nchunk % NBUF == 0 and nchunk > NBUF
    mesh = plsc.VectorSubcoreMesh(core_axis_name="c", subcore_axis_name="s")
    out_t = jax.ShapeDtypeStruct((NW, nchunk, CHUNK, EMBED), jnp.float32)

    @functools.partial(
        pl.kernel,
        mesh=mesh,
        out_type=(out_t, out_t),
        compiler_params=pltpu.CompilerParams(use_tc_tiling_on_sc=False),
        scratch_types=[
            pltpu.VMEM((nchunk, CHUNK), jnp.int32),
            pltpu.VMEM((NBUF, CHUNK, EMBED), jnp.float32),
        ]
        + [pltpu.SemaphoreType.DMA] * (2 * NBUF),
    )
    def gather2(num_idx, unit_idx, num_tab, unit_tab, out_num, out_unit,
                idx_v, rows_v, *sems):
        wid = lax.axis_index("s") * NUM_CORES + lax.axis_index("c")
        sem_g = sems[:NBUF]
        sem_w = sems[NBUF:]

        def fire(tab, b, c):
            pltpu.async_copy(tab.at[idx_v.at[c]], rows_v.at[b], sem_g[b])

        def drain(tab, b, c):
            pltpu.make_async_copy(
                tab.at[idx_v.at[c]], rows_v.at[b], sem_g[b]).wait()

        def put(out, b, c):
            pltpu.async_copy(rows_v.at[b], out.at[wid, c], sem_w[b])

        def put_wait(out, b, c):
            pltpu.make_async_copy(rows_v.at[b], out.at[wid, c], sem_w[b]).wait()

        def run_table(idx_hbm, tab, out):
            pltpu.sync_copy(idx_hbm.at[pl.ds(wid * nchunk, nchunk)], idx_v)
            for c in range(NBUF - 1):
                fire(tab, c, c)

            def step(i, carry):
                for b in range(NBUF):
                    c = i * NBUF + b
                    drain(tab, b, c)
                    put(out, b, c)
                    bf = (b + NBUF - 1) % NBUF

                    @pl.when(c + NBUF - 1 < nchunk)
                    def _():
                        @pl.when(c >= 1)
                        def _():
                            put_wait(out, bf, c - 1)

                        fire(tab, bf, c + NBUF - 1)
                return carry

            lax.fori_loop(0, nchunk // NBUF, step, 0)
            for b in range(NBUF):
                put_wait(out, b, nchunk - NBUF + b)

        run_table(num_idx, num_tab, out_num)
        run_table(unit_idx, unit_tab, out_unit)

    return gather2


def kernel(num_tokens, unit_tokens, num_table, unit_table):
    B, S = num_tokens.shape
    rows = B * S
    assert rows % (NW * CHUNK) == 0
    nchunk = rows // (NW * CHUNK)
    ni = num_tokens.reshape(NW * nchunk, CHUNK).astype(jnp.int32)
    ui = unit_tokens.reshape(NW * nchunk, CHUNK).astype(jnp.int32)
    out_num, out_unit = _make_gather2(nchunk)(ni, ui, num_table, unit_table)
    return (out_num.reshape(B, S, EMBED), out_unit.reshape(B, S, EMBED))


# speedup vs baseline: 2.0496x; 1.0520x over previous
import functools

import jax
import jax.numpy as jnp
from jax import lax
from jax.experimental import pallas as pl
from jax.experimental.pallas import tpu as pltpu
from jax.experimental.pallas import tpu_sc as plsc

EMBED = 64
NUM_CORES = 2
NUM_SUBCORES = 16
NW = NUM_CORES * NUM_SUBCORES
CHUNK = 128
NBUF = 5


@functools.cache
def _make_gather1(nchunk):
    assert nchunk % NBUF == 0 and nchunk > NBUF
    mesh = plsc.VectorSubcoreMesh(core_axis_name="c", subcore_axis_name="s")
    out_t = jax.ShapeDtypeStruct((NW, nchunk, CHUNK, EMBED), jnp.float32)

    @functools.partial(
        pl.kernel,
        mesh=mesh,
        out_type=out_t,
        compiler_params=pltpu.CompilerParams(use_tc_tiling_on_sc=False),
        scratch_types=[
            pltpu.VMEM((nchunk, CHUNK), jnp.int32),
            pltpu.VMEM((NBUF, CHUNK, EMBED), jnp.float32),
        ]
        + [pltpu.SemaphoreType.DMA] * (2 * NBUF),
    )
    def gather1(tok_idx, tab_in, out_ref, idx_v, rows_v, *sems):
        wid = lax.axis_index("s") * NUM_CORES + lax.axis_index("c")
        sem_g = sems[:NBUF]
        sem_w = sems[NBUF:]

        def fire(tab, b, c):
            pltpu.async_copy(tab.at[idx_v.at[c]], rows_v.at[b], sem_g[b])

        def drain(tab, b, c):
            pltpu.make_async_copy(
                tab.at[idx_v.at[c]], rows_v.at[b], sem_g[b]).wait()

        def put(out, b, c):
            pltpu.async_copy(rows_v.at[b], out.at[wid, c], sem_w[b])

        def put_wait(out, b, c):
            pltpu.make_async_copy(rows_v.at[b], out.at[wid, c], sem_w[b]).wait()

        def run_table(idx_hbm, tab, out):
            pltpu.sync_copy(idx_hbm.at[pl.ds(wid * nchunk, nchunk)], idx_v)
            for c in range(NBUF - 1):
                fire(tab, c, c)

            def step(i, carry):
                for b in range(NBUF):
                    c = i * NBUF + b
                    drain(tab, b, c)
                    put(out, b, c)
                    bf = (b + NBUF - 1) % NBUF

                    @pl.when(c + NBUF - 1 < nchunk)
                    def _():
                        @pl.when(c >= 1)
                        def _():
                            put_wait(out, bf, c - 1)

                        fire(tab, bf, c + NBUF - 1)
                return carry

            lax.fori_loop(0, nchunk // NBUF, step, 0)
            for b in range(NBUF):
                put_wait(out, b, nchunk - NBUF + b)

        run_table(tok_idx, tab_in, out_ref)

    return gather1


def kernel(num_tokens, unit_tokens, num_table, unit_table):
    B, S = num_tokens.shape
    rows = B * S
    assert rows % (NW * CHUNK) == 0
    nchunk = rows // (NW * CHUNK)
    ni = num_tokens.reshape(NW * nchunk, CHUNK).astype(jnp.int32)
    ui = unit_tokens.reshape(NW * nchunk, CHUNK).astype(jnp.int32)
    g = _make_gather1(nchunk)
    out_num = g(ni, num_table)
    out_unit = g(ui, unit_table)
    return (out_num.reshape(B, S, EMBED), out_unit.reshape(B, S, EMBED))
